# baseline (speedup 1.0000x reference)
import functools
import jax
import jax.numpy as jnp
from jax import lax
from jax.experimental import pallas as pl
from jax.experimental.pallas import tpu as pltpu
from jax.experimental.pallas import tpu_sc as plsc

B, L, D, V = 4, 2048, 768, 30522
BH = B // 2
NC, NS, LANES = 2, 16, 16
NW = NC * NS
C = L // NW
BR = 512


def _gather_body(x_hbm, word_hbm, out_hbm,
                 idx0, idx1, buf0, buf1, gs0, gs1, ws0, ws1):
    wid = lax.axis_index("s") * NC + lax.axis_index("c")
    l0 = wid * C
    idx = (idx0, idx1)
    buf = (buf0, buf1)
    gsem = (gs0, gs1)
    wsem = (ws0, ws1)

    pltpu.sync_copy(x_hbm.at[0, pl.ds(l0, C)], idx0)
    gathers = [pltpu.async_copy(word_hbm.at[idx0], buf0, gs0)]
    writes = [None, None]
    for b in range(BH):
        p = b % 2
        q = (b + 1) % 2
        if b + 1 < BH:
            pltpu.sync_copy(x_hbm.at[b + 1, pl.ds(l0, C)], idx[q])
            if writes[q] is not None:
                writes[q].wait()
                writes[q] = None
            gathers.append(pltpu.async_copy(word_hbm.at[idx[q]], buf[q],
                                            gsem[q]))
        gathers[b].wait()
        writes[p] = pltpu.async_copy(buf[p],
                                     out_hbm.at[b, pl.ds(l0, C)], wsem[p])
    for w in writes:
        if w is not None:
            w.wait()


_mesh = plsc.VectorSubcoreMesh(core_axis_name="c", subcore_axis_name="s",
                               num_cores=NC, num_subcores=NS)

_sc_gather_half = functools.partial(
    pl.kernel,
    out_type=jax.ShapeDtypeStruct((BH, L, D), jnp.float32),
    mesh=_mesh,
    scratch_types=[
        pltpu.VMEM((C,), jnp.int32),
        pltpu.VMEM((C,), jnp.int32),
        pltpu.VMEM((C, D), jnp.float32),
        pltpu.VMEM((C, D), jnp.float32),
        pltpu.SemaphoreType.DMA,
        pltpu.SemaphoreType.DMA,
        pltpu.SemaphoreType.DMA,
        pltpu.SemaphoreType.DMA,
    ],
)(_gather_body)


def _ln_body(g_ref, seg_ref, pos_ref, tok_ref, gamma_ref, beta_ref, o_ref,
             *maybe_prev):
    segf = seg_ref[0].astype(jnp.float32)[..., None]
    tok0 = tok_ref[0, :]
    tokd = tok_ref[1, :] - tok0
    emb = (g_ref[...] + pos_ref[...][None]
           + (tok0[None, None, :] + segf * tokd[None, None, :]))
    mean = jnp.mean(emb, axis=-1, keepdims=True)
    cent = emb - mean
    var = jnp.mean(cent * cent, axis=-1, keepdims=True)
    rstd = lax.rsqrt(var + 1e-12)
    o_ref[...] = (cent * rstd * gamma_ref[...][None, None, :]
                  + beta_ref[...][None, None, :])


def _ln_body_alias(g_ref, seg_ref, pos_ref, tok_ref, gamma_ref, beta_ref,
                   prev_ref, o_ref):
    _ln_body(g_ref, seg_ref, pos_ref, tok_ref, gamma_ref, beta_ref, o_ref)


def _make_tc_ln(h, alias):
    in_specs = [
        pl.BlockSpec((BH, BR, D), lambda i: (0, i, 0)),
        pl.BlockSpec((1, BH, BR), lambda i: (h, 0, i)),
        pl.BlockSpec((BR, D), lambda i: (i, 0)),
        pl.BlockSpec((8, D), lambda i: (0, 0)),
        pl.BlockSpec((D,), lambda i: (0,)),
        pl.BlockSpec((D,), lambda i: (0,)),
    ]
    kwargs = {}
    body = _ln_body
    if alias:
        in_specs.append(pl.BlockSpec(memory_space=pltpu.HBM))
        kwargs["input_output_aliases"] = {6: 0}
        body = _ln_body_alias
    return pl.pallas_call(
        body,
        grid=(L // BR,),
        in_specs=in_specs,
        out_specs=pl.BlockSpec((BH, BR, D), lambda i: (h, i, 0)),
        out_shape=jax.ShapeDtypeStruct((B, L, D), jnp.float32),
        **kwargs,
    )


_tc_ln0 = _make_tc_ln(0, alias=False)
_tc_ln1 = _make_tc_ln(1, alias=True)


@jax.jit
def kernel(x, segs, word_emb, pos_emb, tok_emb, gamma, beta):
    xi = x.astype(jnp.int32)
    si = segs.astype(jnp.int32).reshape(2, BH, L)
    g0 = _sc_gather_half(xi[:BH], word_emb)
    g1 = _sc_gather_half(xi[BH:], word_emb)
    o0 = _tc_ln0(g0, si, pos_emb, tok_emb, gamma, beta)
    o1 = _tc_ln1(g1, si, pos_emb, tok_emb, gamma, beta, o0)
    return o1
